# baseline (device time: 85498 ns/iter reference)
import jax
import jax.numpy as jnp
from jax import lax
from jax.experimental import pallas as pl
from jax.experimental.pallas import tpu as pltpu

N = 8
S = 1024
H, Dh, Dr = 16, 128, 32
HPD = H // N
CW = HPD * Dh
DC = 128
NB = 4
BS = S // NB
WCH = 16
WR = 2048 // WCH

BF = jnp.bfloat16
F32 = jnp.float32


def _mla_fused(x, Wdkv, Wuk, Wuv, Wq, Wqr0, Wqr1, Wkr, Wo):
    scale = (Dh + Dr) ** -0.5

    def body(x_ref, wdkv_ref, wuk_ref, wuv_ref, wq_ref, wqr0_ref,
             wqr1_ref, wkr_ref, wo_ref, out_ref,
             xbf, wdkvbf, wukbf, wuvbf, wqstg, wqbf,
             cbf, cbuf, wkbuf, wvbuf, kf32, vf32, kbf, vbf, qbf,
             qr0, qr1, krb, wobf, wostg, outstg, obuf,
             c_ss, wk_ss, wv_ss, c_rs, wk_rs, wv_rs, o_ss, o_rs,
             wo_sem, wq_sem, ost_sem):
        me = lax.axis_index("i")
        g = lax.rem(me + 1, N)

        wqdma = pltpu.make_async_copy(
            wq_ref.at[:, pl.ds(g * CW, CW)], wqstg, wq_sem)
        wqdma.start()
        xbf[...] = x_ref[0].astype(BF)
        wdkvbf[...] = wdkv_ref[...].astype(BF)
        wukbf[...] = wuk_ref[...].astype(BF)
        wuvbf[...] = wuv_ref[...].astype(BF)
        cbf[...] = jnp.dot(xbf[...], wdkvbf[...],
                           preferred_element_type=F32).astype(BF)

        descs = []
        for k in range(1, N):
            j = lax.rem(me + k, N)
            gj = lax.rem(j + 1, N)
            d_c = pltpu.make_async_remote_copy(
                src_ref=cbf, dst_ref=cbuf.at[me],
                send_sem=c_ss.at[k - 1], recv_sem=c_rs.at[me],
                device_id=(j,), device_id_type=pl.DeviceIdType.MESH,
            )
            d_k = pltpu.make_async_remote_copy(
                src_ref=wukbf.at[:, pl.ds(gj * CW, CW)],
                dst_ref=wkbuf.at[me],
                send_sem=wk_ss.at[k - 1], recv_sem=wk_rs.at[me],
                device_id=(j,), device_id_type=pl.DeviceIdType.MESH,
            )
            d_v = pltpu.make_async_remote_copy(
                src_ref=wuvbf.at[:, pl.ds(gj * CW, CW)],
                dst_ref=wvbuf.at[me],
                send_sem=wv_ss.at[k - 1], recv_sem=wv_rs.at[me],
                device_id=(j,), device_id_type=pl.DeviceIdType.MESH,
            )
            d_c.start()
            d_k.start()
            d_v.start()
            descs.extend([d_c, d_k, d_v])

        for ch in range(WCH):
            wdma = pltpu.make_async_copy(
                wo_ref.at[pl.ds(ch * WR, WR), :], wostg, wo_sem)
            wdma.start()
            wdma.wait()
            wobf[pl.ds(ch * WR, WR), :] = wostg[...].astype(BF)

        wqdma.wait()
        wqbf[...] = wqstg[...].astype(BF)
        qbf[...] = jnp.dot(xbf[...], wqbf[...],
                           preferred_element_type=F32).astype(BF)
        qr0[...] = jnp.dot(xbf[...], wqr0_ref[...],
                           preferred_element_type=F32).astype(BF)
        qr1[...] = jnp.dot(xbf[...], wqr1_ref[...],
                           preferred_element_type=F32).astype(BF)
        krb[...] = jnp.dot(xbf[...], wkr_ref[...],
                           preferred_element_type=F32).astype(BF)

        kf32[...] = jnp.dot(cbf[...], wukbf[:, pl.ds(g * CW, CW)],
                            preferred_element_type=F32)
        vf32[...] = jnp.dot(cbf[...], wuvbf[:, pl.ds(g * CW, CW)],
                            preferred_element_type=F32)

        for k in range(1, N):
            j = lax.rem(me + k, N)
            wc = pltpu.make_async_remote_copy(
                src_ref=cbf, dst_ref=cbuf.at[j],
                send_sem=c_ss.at[k - 1], recv_sem=c_rs.at[j],
                device_id=(j,), device_id_type=pl.DeviceIdType.MESH,
            )
            wk_ = pltpu.make_async_remote_copy(
                src_ref=wukbf.at[:, pl.ds(0, CW)], dst_ref=wkbuf.at[j],
                send_sem=wk_ss.at[k - 1], recv_sem=wk_rs.at[j],
                device_id=(j,), device_id_type=pl.DeviceIdType.MESH,
            )
            wv_ = pltpu.make_async_remote_copy(
                src_ref=wuvbf.at[:, pl.ds(0, CW)], dst_ref=wvbuf.at[j],
                send_sem=wv_ss.at[k - 1], recv_sem=wv_rs.at[j],
                device_id=(j,), device_id_type=pl.DeviceIdType.MESH,
            )
            wc.wait_recv()
            wk_.wait_recv()
            kf32[...] = kf32[...] + jnp.dot(cbuf[j], wkbuf[j],
                                            preferred_element_type=F32)
            wv_.wait_recv()
            vf32[...] = vf32[...] + jnp.dot(cbuf[j], wvbuf[j],
                                            preferred_element_type=F32)

        kbf[...] = kf32[...].astype(BF)
        vbf[...] = vf32[...].astype(BF)

        for b in range(NB):
            rows = pl.ds(b * BS, BS)
            for h in range(HPD):
                cols = pl.ds(h * Dh, Dh)
                s = lax.dot_general(
                    qbf[rows, cols], kbf[:, cols],
                    (((1,), (1,)), ((), ())),
                    preferred_element_type=F32,
                )
                qr_ref = qr0 if h == 0 else qr1
                s = s + lax.dot_general(
                    qr_ref[rows, :], krb[...],
                    (((1,), (1,)), ((), ())),
                    preferred_element_type=F32,
                )
                s = s * scale
                m = jnp.max(s, axis=1, keepdims=True)
                p = jnp.exp(s - m)
                p = (p / jnp.sum(p, axis=1, keepdims=True)).astype(BF)
                ob = jnp.dot(p, vbf[:, cols],
                             preferred_element_type=F32)
                obuf[rows, pl.ds(g * CW + h * Dh, Dh)] = ob.astype(BF)

            for k in range(1, N):
                j = lax.rem(me + k, N)
                d = pltpu.make_async_remote_copy(
                    src_ref=obuf.at[rows, pl.ds(g * CW, CW)],
                    dst_ref=obuf.at[rows, pl.ds(g * CW, CW)],
                    send_sem=o_ss.at[k - 1, b],
                    recv_sem=o_rs.at[me, b],
                    device_id=(j,), device_id_type=pl.DeviceIdType.MESH,
                )
                d.start()
                descs.append(d)

        for b in range(NB):
            rows = pl.ds(b * BS, BS)
            for k in range(1, N):
                j = lax.rem(me + k, N)
                gj = lax.rem(j + 1, N)
                w = pltpu.make_async_remote_copy(
                    src_ref=obuf.at[rows, pl.ds(g * CW, CW)],
                    dst_ref=obuf.at[rows, pl.ds(gj * CW, CW)],
                    send_sem=o_ss.at[k - 1, b],
                    recv_sem=o_rs.at[j, b],
                    device_id=(j,), device_id_type=pl.DeviceIdType.MESH,
                )
                w.wait_recv()
            outstg[...] = jnp.dot(
                obuf[rows, :], wobf[...], preferred_element_type=F32)
            odma = pltpu.make_async_copy(
                outstg, out_ref.at[0, rows, :], ost_sem)
            odma.start()
            odma.wait()

        for d in descs:
            d.wait_send()

    return pl.pallas_call(
        body,
        out_shape=jax.ShapeDtypeStruct((1, S, Wo.shape[1]), F32),
        in_specs=(
            [pl.BlockSpec(memory_space=pltpu.VMEM)] * 4
            + [pl.BlockSpec(memory_space=pltpu.MemorySpace.HBM)]
            + [pl.BlockSpec(memory_space=pltpu.VMEM)] * 3
            + [pl.BlockSpec(memory_space=pltpu.MemorySpace.HBM)]
        ),
        out_specs=pl.BlockSpec(memory_space=pltpu.MemorySpace.HBM),
        scratch_shapes=[
            pltpu.VMEM((S, 2048), BF),
            pltpu.VMEM((2048, DC), BF),
            pltpu.VMEM((DC, 2048), BF),
            pltpu.VMEM((DC, 2048), BF),
            pltpu.VMEM((2048, CW), F32),
            pltpu.VMEM((2048, CW), BF),
            pltpu.VMEM((S, DC), BF),
            pltpu.VMEM((N, S, DC), BF),
            pltpu.VMEM((N, DC, CW), BF),
            pltpu.VMEM((N, DC, CW), BF),
            pltpu.VMEM((S, CW), F32),
            pltpu.VMEM((S, CW), F32),
            pltpu.VMEM((S, CW), BF),
            pltpu.VMEM((S, CW), BF),
            pltpu.VMEM((S, CW), BF),
            pltpu.VMEM((S, Dr), BF),
            pltpu.VMEM((S, Dr), BF),
            pltpu.VMEM((S, Dr), BF),
            pltpu.VMEM((2048, 2048), BF),
            pltpu.VMEM((WR, 2048), F32),
            pltpu.VMEM((BS, 2048), F32),
            pltpu.VMEM((S, N * CW), BF),
            pltpu.SemaphoreType.DMA((N - 1,)),
            pltpu.SemaphoreType.DMA((N - 1,)),
            pltpu.SemaphoreType.DMA((N - 1,)),
            pltpu.SemaphoreType.DMA((N,)),
            pltpu.SemaphoreType.DMA((N,)),
            pltpu.SemaphoreType.DMA((N,)),
            pltpu.SemaphoreType.DMA((N - 1, NB)),
            pltpu.SemaphoreType.DMA((N, NB)),
            pltpu.SemaphoreType.DMA,
            pltpu.SemaphoreType.DMA,
            pltpu.SemaphoreType.DMA,
        ],
    )(x, Wdkv, Wuk, Wuv, Wq, Wqr0, Wqr1, Wkr, Wo)


def kernel(x, Wdkv, Wuk, Wuv, Wq, Wqr, Wkr, Wo):
    g = lax.rem(lax.axis_index("i") + 1, N)

    Wqr_loc = lax.dynamic_slice(Wqr, (0, g * HPD * Dr),
                                (Wqr.shape[0], HPD * Dr))

    return _mla_fused(
        x, Wdkv, Wuk, Wuv, Wq, Wqr_loc[:, :Dr].astype(BF),
        Wqr_loc[:, Dr:].astype(BF), Wkr.astype(BF), Wo)


# device time: 79305 ns/iter; 1.0781x vs baseline; 1.0781x over previous
import jax
import jax.numpy as jnp
from jax import lax
from jax.experimental import pallas as pl
from jax.experimental.pallas import tpu as pltpu

N = 8
S = 1024
H, Dh, Dr = 16, 128, 32
HPD = H // N
CW = HPD * Dh
DC = 128
NB = 4
BS = S // NB
WCH = 8
WR = 2048 // WCH

BF = jnp.bfloat16
F32 = jnp.float32


def _mla_fused(x, Wdkv, Wuk, Wuv, Wq_loc, Wqr0, Wqr1, Wkr, Wo):
    scale = (Dh + Dr) ** -0.5

    def body(x_ref, wdkv_ref, wuk_ref, wuv_ref, wq_ref, wqr0_ref,
             wqr1_ref, wkr_ref, wo_ref, out_ref,
             xbf, cbf, cbuf, wkbuf, wvbuf, kf32, vf32, kbf, vbf, qbf,
             qr0, qr1, krb, wobf, wostg, outstg, obuf,
             c_ss, wk_ss, wv_ss, c_rs, wk_rs, wv_rs, o_ss, o_rs,
             wo_sem, ost_sem):
        me = lax.axis_index("i")
        g = lax.rem(me + 1, N)

        xbf[...] = x_ref[0].astype(BF)
        cbf[...] = jnp.dot(xbf[...], wdkv_ref[...],
                           preferred_element_type=F32).astype(BF)

        descs = []
        for k in range(1, N):
            j = lax.rem(me + k, N)
            gj = lax.rem(j + 1, N)
            d_c = pltpu.make_async_remote_copy(
                src_ref=cbf, dst_ref=cbuf.at[me],
                send_sem=c_ss.at[k - 1], recv_sem=c_rs.at[me],
                device_id=(j,), device_id_type=pl.DeviceIdType.MESH,
            )
            d_k = pltpu.make_async_remote_copy(
                src_ref=wuk_ref.at[:, pl.ds(gj * CW, CW)],
                dst_ref=wkbuf.at[me],
                send_sem=wk_ss.at[k - 1], recv_sem=wk_rs.at[me],
                device_id=(j,), device_id_type=pl.DeviceIdType.MESH,
            )
            d_v = pltpu.make_async_remote_copy(
                src_ref=wuv_ref.at[:, pl.ds(gj * CW, CW)],
                dst_ref=wvbuf.at[me],
                send_sem=wv_ss.at[k - 1], recv_sem=wv_rs.at[me],
                device_id=(j,), device_id_type=pl.DeviceIdType.MESH,
            )
            d_c.start()
            d_k.start()
            d_v.start()
            descs.extend([d_c, d_k, d_v])

        for ch in range(WCH):
            wdma = pltpu.make_async_copy(
                wo_ref.at[pl.ds(ch * WR, WR), :], wostg, wo_sem)
            wdma.start()
            wdma.wait()
            wobf[pl.ds(ch * WR, WR), :] = wostg[...].astype(BF)

        qbf[...] = jnp.dot(xbf[...], wq_ref[...],
                           preferred_element_type=F32).astype(BF)
        qr0[...] = jnp.dot(xbf[...], wqr0_ref[...],
                           preferred_element_type=F32).astype(BF)
        qr1[...] = jnp.dot(xbf[...], wqr1_ref[...],
                           preferred_element_type=F32).astype(BF)
        krb[...] = jnp.dot(xbf[...], wkr_ref[...],
                           preferred_element_type=F32).astype(BF)

        kf32[...] = jnp.dot(cbf[...], wuk_ref[:, pl.ds(g * CW, CW)],
                            preferred_element_type=F32)
        vf32[...] = jnp.dot(cbf[...], wuv_ref[:, pl.ds(g * CW, CW)],
                            preferred_element_type=F32)

        for k in range(1, N):
            j = lax.rem(me + k, N)
            wc = pltpu.make_async_remote_copy(
                src_ref=cbf, dst_ref=cbuf.at[j],
                send_sem=c_ss.at[k - 1], recv_sem=c_rs.at[j],
                device_id=(j,), device_id_type=pl.DeviceIdType.MESH,
            )
            wk_ = pltpu.make_async_remote_copy(
                src_ref=wuk_ref.at[:, pl.ds(0, CW)], dst_ref=wkbuf.at[j],
                send_sem=wk_ss.at[k - 1], recv_sem=wk_rs.at[j],
                device_id=(j,), device_id_type=pl.DeviceIdType.MESH,
            )
            wv_ = pltpu.make_async_remote_copy(
                src_ref=wuv_ref.at[:, pl.ds(0, CW)], dst_ref=wvbuf.at[j],
                send_sem=wv_ss.at[k - 1], recv_sem=wv_rs.at[j],
                device_id=(j,), device_id_type=pl.DeviceIdType.MESH,
            )
            wc.wait_recv()
            wk_.wait_recv()
            kf32[...] = kf32[...] + jnp.dot(cbuf[j], wkbuf[j],
                                            preferred_element_type=F32)
            wv_.wait_recv()
            vf32[...] = vf32[...] + jnp.dot(cbuf[j], wvbuf[j],
                                            preferred_element_type=F32)

        kbf[...] = kf32[...].astype(BF)
        vbf[...] = vf32[...].astype(BF)

        for b in range(NB):
            rows = pl.ds(b * BS, BS)
            for h in range(HPD):
                cols = pl.ds(h * Dh, Dh)
                s = lax.dot_general(
                    qbf[rows, cols], kbf[:, cols],
                    (((1,), (1,)), ((), ())),
                    preferred_element_type=F32,
                )
                qr_ref = qr0 if h == 0 else qr1
                s = s + lax.dot_general(
                    qr_ref[rows, :], krb[...],
                    (((1,), (1,)), ((), ())),
                    preferred_element_type=F32,
                )
                s = s * scale
                m = jnp.max(s, axis=1, keepdims=True)
                p = jnp.exp(s - m)
                p = (p / jnp.sum(p, axis=1, keepdims=True)).astype(BF)
                ob = jnp.dot(p, vbf[:, cols],
                             preferred_element_type=F32)
                obuf[rows, pl.ds(g * CW + h * Dh, Dh)] = ob.astype(BF)

            for k in range(1, N):
                j = lax.rem(me + k, N)
                d = pltpu.make_async_remote_copy(
                    src_ref=obuf.at[rows, pl.ds(g * CW, CW)],
                    dst_ref=obuf.at[rows, pl.ds(g * CW, CW)],
                    send_sem=o_ss.at[k - 1, b],
                    recv_sem=o_rs.at[me, b],
                    device_id=(j,), device_id_type=pl.DeviceIdType.MESH,
                )
                d.start()
                descs.append(d)

        for b in range(NB):
            rows = pl.ds(b * BS, BS)
            for k in range(1, N):
                j = lax.rem(me + k, N)
                gj = lax.rem(j + 1, N)
                w = pltpu.make_async_remote_copy(
                    src_ref=obuf.at[rows, pl.ds(g * CW, CW)],
                    dst_ref=obuf.at[rows, pl.ds(gj * CW, CW)],
                    send_sem=o_ss.at[k - 1, b],
                    recv_sem=o_rs.at[j, b],
                    device_id=(j,), device_id_type=pl.DeviceIdType.MESH,
                )
                w.wait_recv()
            outstg[...] = jnp.dot(
                obuf[rows, :], wobf[...], preferred_element_type=F32)
            odma = pltpu.make_async_copy(
                outstg, out_ref.at[0, rows, :], ost_sem)
            odma.start()
            odma.wait()

        for d in descs:
            d.wait_send()

    return pl.pallas_call(
        body,
        out_shape=jax.ShapeDtypeStruct((1, S, Wo.shape[1]), F32),
        in_specs=(
            [pl.BlockSpec(memory_space=pltpu.VMEM)] * 8
            + [pl.BlockSpec(memory_space=pltpu.MemorySpace.HBM)]
        ),
        out_specs=pl.BlockSpec(memory_space=pltpu.MemorySpace.HBM),
        scratch_shapes=[
            pltpu.VMEM((S, 2048), BF),
            pltpu.VMEM((S, DC), BF),
            pltpu.VMEM((N, S, DC), BF),
            pltpu.VMEM((N, DC, CW), BF),
            pltpu.VMEM((N, DC, CW), BF),
            pltpu.VMEM((S, CW), F32),
            pltpu.VMEM((S, CW), F32),
            pltpu.VMEM((S, CW), BF),
            pltpu.VMEM((S, CW), BF),
            pltpu.VMEM((S, CW), BF),
            pltpu.VMEM((S, Dr), BF),
            pltpu.VMEM((S, Dr), BF),
            pltpu.VMEM((S, Dr), BF),
            pltpu.VMEM((2048, 2048), BF),
            pltpu.VMEM((WR, 2048), F32),
            pltpu.VMEM((BS, 2048), F32),
            pltpu.VMEM((S, N * CW), BF),
            pltpu.SemaphoreType.DMA((N - 1,)),
            pltpu.SemaphoreType.DMA((N - 1,)),
            pltpu.SemaphoreType.DMA((N - 1,)),
            pltpu.SemaphoreType.DMA((N,)),
            pltpu.SemaphoreType.DMA((N,)),
            pltpu.SemaphoreType.DMA((N,)),
            pltpu.SemaphoreType.DMA((N - 1, NB)),
            pltpu.SemaphoreType.DMA((N, NB)),
            pltpu.SemaphoreType.DMA,
            pltpu.SemaphoreType.DMA,
        ],
    )(x, Wdkv, Wuk, Wuv, Wq_loc, Wqr0, Wqr1, Wkr, Wo)


def kernel(x, Wdkv, Wuk, Wuv, Wq, Wqr, Wkr, Wo):
    g = lax.rem(lax.axis_index("i") + 1, N)

    Wq_loc = lax.dynamic_slice(Wq, (0, g * CW), (Wq.shape[0], CW))
    Wqr_loc = lax.dynamic_slice(Wqr, (0, g * HPD * Dr),
                                (Wqr.shape[0], HPD * Dr))

    return _mla_fused(
        x, Wdkv.astype(BF), Wuk.astype(BF), Wuv.astype(BF),
        Wq_loc.astype(BF), Wqr_loc[:, :Dr].astype(BF),
        Wqr_loc[:, Dr:].astype(BF), Wkr.astype(BF), Wo)
